# baseline (speedup 1.0000x reference)
import jax
import jax.numpy as jnp
from jax.experimental import pallas as pl

N = 10000
G = 64
EPS = 1e-5


def _gcn_conv(x, src, dst, W, b, n):
    xw = x @ W
    loop = jnp.arange(n, dtype=src.dtype)
    s = jnp.concatenate([src, loop])
    d = jnp.concatenate([dst, loop])
    deg = jnp.zeros((n,), dtype=x.dtype).at[d].add(1.0)
    dinv = jax.lax.rsqrt(deg)
    norm = dinv[s] * dinv[d]
    msg = xw[s] * norm[:, None]
    out = jnp.zeros((n, xw.shape[1]), dtype=x.dtype).at[d].add(msg)
    return out + b


def _bn(x, gamma, beta):
    mu = jnp.mean(x, axis=0)
    var = jnp.var(x, axis=0)
    return (x - mu) * jax.lax.rsqrt(var + EPS) * gamma + beta


def _copy_kernel(x_ref, o_ref):
    o_ref[...] = x_ref[...]


def kernel(x, edge_index, batch, W_in, b_in, W1, b1, W2, b2, g1, be1, g2, be2,
           Wl1, bl1, gf1, bef1, Wl2, bl2, gf2, bef2, Wl3, bl3):
    src, dst = edge_index[0], edge_index[1]
    h = x @ W_in + b_in
    h = _gcn_conv(h, src, dst, W1, b1, N)
    h = jax.nn.relu(_bn(h, g1, be1))
    h = _gcn_conv(h, src, dst, W2, b2, N)
    h = jax.nn.relu(_bn(h, g2, be2))
    ones = jnp.ones((N,), jnp.float32)
    cnt = jax.ops.segment_sum(ones, batch, num_segments=G)
    mean_pool = jax.ops.segment_sum(h, batch, num_segments=G) / jnp.maximum(cnt, 1.0)[:, None]
    max_pool = jax.ops.segment_max(h, batch, num_segments=G)
    max_pool = jnp.where(cnt[:, None] > 0, max_pool, 0.0)
    hg = jnp.concatenate([mean_pool, max_pool], axis=1)
    hg = jax.nn.relu(_bn(hg @ Wl1 + bl1, gf1, bef1))
    hg = jax.nn.relu(_bn(hg @ Wl2 + bl2, gf2, bef2))
    out = hg @ Wl3 + bl3
    return pl.pallas_call(
        _copy_kernel,
        out_shape=jax.ShapeDtypeStruct(out.shape, out.dtype),
    )(out)


# speedup vs baseline: 6.5687x; 6.5687x over previous
import functools

import jax
import jax.numpy as jnp
from jax import lax
from jax.experimental import pallas as pl
from jax.experimental.pallas import tpu as pltpu
from jax.experimental.pallas import tpu_sc as plsc

N = 10000
E = 320000
H = 128
G = 64
EPS = 1e-5

NC, NS = 2, 16
NW = NC * NS
NP = 10240
K = 128
CH = 80
RPT = NP // NS
EPW = NW * CH * K // NW
BLK = 2048
NB = NP // BLK

@functools.cache
def _mesh():
    return plsc.VectorSubcoreMesh(core_axis_name="c", subcore_axis_name="s",
                                  num_cores=NC, num_subcores=NS)


def _sc_deg_body(dst_hbm, degp_hbm, idx_v, cnt_v, red_v, wb_v, sh):
    c = lax.axis_index("c")
    s = lax.axis_index("s")
    wid = c * NS + s
    row0 = s * RPT
    ones = jnp.ones((16,), jnp.float32)
    zeros = jnp.zeros((16,), jnp.float32)

    def zstep(j, carry):
        cnt_v[pl.ds(j * 16, 16)] = zeros
        return carry

    lax.fori_loop(0, NP // 16, zstep, 0)
    pltpu.sync_copy(dst_hbm.at[pl.ds(wid * EPW, EPW)], idx_v)

    def astep(j, carry):
        idx16 = idx_v[pl.ds(j * 16, 16)]
        plsc.addupdate_scatter(cnt_v, [idx16], ones)
        return carry

    lax.fori_loop(0, EPW // 16, astep, 0)
    pltpu.sync_copy(cnt_v, sh.at[pl.ds(s * NP, NP)])
    plsc.subcore_barrier()
    for t in range(NS):
        pltpu.sync_copy(sh.at[pl.ds(t * NP + row0, RPT)], red_v.at[t])

    def rstep(j, carry):
        acc = red_v[0, pl.ds(j * 16, 16)]
        for t in range(1, NS):
            acc = acc + red_v[t, pl.ds(j * 16, 16)]
        wb_v[pl.ds(j * 16, 16)] = acc
        return carry

    lax.fori_loop(0, RPT // 16, rstep, 0)
    pltpu.sync_copy(wb_v, degp_hbm.at[pl.ds(c * NP + row0, RPT)])


def _DEG(dst1):
    return pl.kernel(
        _sc_deg_body,
        out_type=jax.ShapeDtypeStruct((NC * NP,), jnp.float32),
        mesh=_mesh(),
        compiler_params=pltpu.CompilerParams(needs_layout_passes=False),
        scratch_types=[
            pltpu.VMEM((EPW,), jnp.int32),
            pltpu.VMEM((NP,), jnp.float32),
            pltpu.VMEM((NS, RPT), jnp.float32),
            pltpu.VMEM((RPT,), jnp.float32),
            pltpu.VMEM_SHARED((NS * NP,), jnp.float32),
        ],
    )(dst1)


def _sc_agg_body(y_hbm, src_hbm, dst_hbm, zeros_hbm, accp_hbm, src_v, dst_v,
                 rows_v, acc_sh, sem):
    c = lax.axis_index("c")
    s = lax.axis_index("s")
    wid = c * NS + s
    row0 = s * RPT
    pltpu.sync_copy(zeros_hbm, rows_v)
    for k in range(RPT // K):
        pltpu.sync_copy(rows_v, acc_sh.at[pl.ds(row0 + k * K, K)])
    plsc.subcore_barrier()

    def step(j, carry):
        base = (wid * CH + j) * K
        pltpu.sync_copy(src_hbm.at[pl.ds(base, K)], src_v)
        pltpu.sync_copy(dst_hbm.at[pl.ds(base, K)], dst_v)
        pltpu.async_copy(y_hbm.at[src_v], rows_v, sem).wait()
        pltpu.sync_copy(rows_v, acc_sh.at[dst_v], add=True)
        return carry

    lax.fori_loop(0, CH, step, 0)
    plsc.subcore_barrier()
    for k in range(RPT // K):
        pltpu.sync_copy(acc_sh.at[pl.ds(row0 + k * K, K)], rows_v)
        pltpu.sync_copy(rows_v, accp_hbm.at[pl.ds(c * NP + row0 + k * K, K)])


def _AGG(y, src1, dst1, zrows):
    return pl.kernel(
        _sc_agg_body,
        out_type=jax.ShapeDtypeStruct((NC * NP, H), jnp.float32),
        mesh=_mesh(),
        scratch_types=[
            pltpu.VMEM((K,), jnp.int32),
            pltpu.VMEM((K,), jnp.int32),
            pltpu.VMEM((K, H), jnp.float32),
            pltpu.VMEM_SHARED((NP, H), jnp.float32),
            pltpu.SemaphoreType.DMA,
        ],
    )(y, src1, dst1, zrows)


def _row_mask(i):
    rows = i * BLK + lax.broadcasted_iota(jnp.int32, (BLK, 1), 0)
    return (rows < N).astype(jnp.float32)


def _tc_in_body(x_ref, degp_ref, wi_ref, bi_ref, w1_ref, y_ref, dinv_ref):
    i = pl.program_id(0)
    deg = degp_ref[0] + degp_ref[1] + 1.0
    dinv = lax.rsqrt(deg)
    h0 = jnp.dot(x_ref[...], wi_ref[...],
                 preferred_element_type=jnp.float32) + bi_ref[...]
    xw = jnp.dot(h0, w1_ref[...], preferred_element_type=jnp.float32)
    y_ref[...] = xw * dinv * _row_mask(i)
    dinv_ref[...] = dinv


def _tc_in(xp, degp, w_in, b_in, w1):
    return pl.pallas_call(
        _tc_in_body,
        grid=(NB,),
        in_specs=[
            pl.BlockSpec((BLK, 3), lambda i: (i, 0)),
            pl.BlockSpec((NC, BLK, 1), lambda i: (0, i, 0)),
            pl.BlockSpec((3, H), lambda i: (0, 0)),
            pl.BlockSpec((1, H), lambda i: (0, 0)),
            pl.BlockSpec((H, H), lambda i: (0, 0)),
        ],
        out_specs=[
            pl.BlockSpec((BLK, H), lambda i: (i, 0)),
            pl.BlockSpec((BLK, 1), lambda i: (i, 0)),
        ],
        out_shape=[
            jax.ShapeDtypeStruct((NP, H), jnp.float32),
            jax.ShapeDtypeStruct((NP, 1), jnp.float32),
        ],
    )(xp, degp, w_in, b_in, w1)


def _tc_post_body(accp_ref, y_ref, dinv_ref, b_ref, out_ref, st_ref):
    i = pl.program_id(0)
    acc = accp_ref[0] + accp_ref[1] + y_ref[...]
    out = (acc * dinv_ref[...] + b_ref[...]) * _row_mask(i)
    out_ref[...] = out
    st = jnp.concatenate(
        [jnp.sum(out, 0, keepdims=True), jnp.sum(out * out, 0, keepdims=True)],
        axis=0)

    @pl.when(i == 0)
    def _():
        st_ref[...] = st

    @pl.when(i > 0)
    def _():
        st_ref[...] = st_ref[...] + st


def _tc_post(accp, y, dinv, b):
    return pl.pallas_call(
        _tc_post_body,
        grid=(NB,),
        in_specs=[
            pl.BlockSpec((NC, BLK, H), lambda i: (0, i, 0)),
            pl.BlockSpec((BLK, H), lambda i: (i, 0)),
            pl.BlockSpec((BLK, 1), lambda i: (i, 0)),
            pl.BlockSpec((1, H), lambda i: (0, 0)),
        ],
        out_specs=[
            pl.BlockSpec((BLK, H), lambda i: (i, 0)),
            pl.BlockSpec((2, H), lambda i: (0, 0)),
        ],
        out_shape=[
            jax.ShapeDtypeStruct((NP, H), jnp.float32),
            jax.ShapeDtypeStruct((2, H), jnp.float32),
        ],
    )(accp, y, dinv, b)


def _tc_mid_body(h_ref, st_ref, g_ref, be_ref, w_ref, dinv_ref, y_ref):
    i = pl.program_id(0)
    mu = st_ref[0:1] / N
    var = st_ref[1:2] / N - mu * mu
    sc = g_ref[...] * lax.rsqrt(var + EPS)
    sh = be_ref[...] - mu * sc
    h = jnp.maximum(h_ref[...] * sc + sh, 0.0)
    y = jnp.dot(h, w_ref[...], preferred_element_type=jnp.float32)
    y_ref[...] = y * dinv_ref[...] * _row_mask(i)


def _tc_mid(out1, st, g, be, w2, dinv):
    return pl.pallas_call(
        _tc_mid_body,
        grid=(NB,),
        in_specs=[
            pl.BlockSpec((BLK, H), lambda i: (i, 0)),
            pl.BlockSpec((2, H), lambda i: (0, 0)),
            pl.BlockSpec((1, H), lambda i: (0, 0)),
            pl.BlockSpec((1, H), lambda i: (0, 0)),
            pl.BlockSpec((H, H), lambda i: (0, 0)),
            pl.BlockSpec((BLK, 1), lambda i: (i, 0)),
        ],
        out_specs=pl.BlockSpec((BLK, H), lambda i: (i, 0)),
        out_shape=jax.ShapeDtypeStruct((NP, H), jnp.float32),
    )(out1, st, g, be, w2, dinv)


def _bn_rows(t, gamma, beta):
    mu = jnp.mean(t, 0, keepdims=True)
    var = jnp.mean(t * t, 0, keepdims=True) - mu * mu
    return (t - mu) * lax.rsqrt(var + EPS) * gamma + beta


def _tc_head_body(h_ref, st_ref, g_ref, be_ref, brow_ref, bcol_ref,
                  wl1_ref, bl1_ref, gf1_ref, bef1_ref,
                  wl2_ref, bl2_ref, gf2_ref, bef2_ref,
                  wl3_ref, bl3_ref, o_ref, sum_s, max_s, cnt_s):
    i = pl.program_id(0)
    mu = st_ref[0:1] / N
    var = st_ref[1:2] / N - mu * mu
    sc = g_ref[...] * lax.rsqrt(var + EPS)
    sh = be_ref[...] - mu * sc
    h = jnp.maximum(h_ref[...] * sc + sh, 0.0)

    gid = lax.broadcasted_iota(jnp.int32, (G, 1), 0)
    oht = (brow_ref[...] == gid).astype(jnp.float32)
    psum = jnp.dot(oht, h, preferred_element_type=jnp.float32)
    pcnt = jnp.sum(oht, axis=1, keepdims=True)
    bcol = bcol_ref[...]
    parts = [
        jnp.max(jnp.where(bcol == g, h, -1e30), axis=0, keepdims=True)
        for g in range(G)
    ]
    pmax = jnp.concatenate(parts, axis=0)

    @pl.when(i == 0)
    def _():
        sum_s[...] = psum
        cnt_s[...] = pcnt
        max_s[...] = pmax

    @pl.when(i > 0)
    def _():
        sum_s[...] = sum_s[...] + psum
        cnt_s[...] = cnt_s[...] + pcnt
        max_s[...] = jnp.maximum(max_s[...], pmax)

    @pl.when(i == NB - 1)
    def _():
        cnt = cnt_s[...]
        mean = sum_s[...] / jnp.maximum(cnt, 1.0)
        mx = jnp.where(cnt > 0, max_s[...], 0.0)
        hg = jnp.concatenate([mean, mx], axis=1)
        t = jnp.dot(hg, wl1_ref[...],
                    preferred_element_type=jnp.float32) + bl1_ref[...]
        t = jnp.maximum(_bn_rows(t, gf1_ref[...], bef1_ref[...]), 0.0)
        t = jnp.dot(t, wl2_ref[...],
                    preferred_element_type=jnp.float32) + bl2_ref[...]
        t = jnp.maximum(_bn_rows(t, gf2_ref[...], bef2_ref[...]), 0.0)
        o_ref[...] = jnp.dot(t, wl3_ref[...],
                             preferred_element_type=jnp.float32) + bl3_ref[...]


def _tc_head(out2, st, g, be, brow, bcol, wl1, bl1, gf1, bef1, wl2, bl2, gf2,
             bef2, wl3, bl3):
    full = lambda shape: pl.BlockSpec(shape, lambda i: tuple(0 for _ in shape))
    return pl.pallas_call(
        _tc_head_body,
        grid=(NB,),
        in_specs=[
            pl.BlockSpec((BLK, H), lambda i: (i, 0)),
            full((2, H)),
            full((1, H)),
            full((1, H)),
            pl.BlockSpec((1, BLK), lambda i: (0, i)),
            pl.BlockSpec((BLK, 1), lambda i: (i, 0)),
            full((2 * H, H)),
            full((1, H)),
            full((1, H)),
            full((1, H)),
            full((H, G)),
            full((1, G)),
            full((1, G)),
            full((1, G)),
            full((G, 2)),
            full((1, 2)),
        ],
        out_specs=full((G, 2)),
        out_shape=jax.ShapeDtypeStruct((G, 2), jnp.float32),
        scratch_shapes=[
            pltpu.VMEM((G, H), jnp.float32),
            pltpu.VMEM((G, H), jnp.float32),
            pltpu.VMEM((G, 1), jnp.float32),
        ],
    )(out2, st, g, be, brow, bcol, wl1, bl1, gf1, bef1, wl2, bl2, gf2, bef2,
      wl3, bl3)


def kernel(x, edge_index, batch, W_in, b_in, W1, b1, W2, b2, g1, be1, g2, be2,
           Wl1, bl1, gf1, bef1, Wl2, bl2, gf2, bef2, Wl3, bl3):
    f32 = jnp.float32
    pad_e = NW * CH * K - E
    src1 = jnp.concatenate([edge_index[0], jnp.full((pad_e,), N, jnp.int32)])
    dst1 = jnp.concatenate([edge_index[1], jnp.full((pad_e,), N, jnp.int32)])
    xp = jnp.zeros((NP, 3), f32).at[:N].set(x)
    batchp = jnp.concatenate([batch, jnp.full((NP - N,), G, jnp.int32)])
    brow = batchp.reshape(1, NP)
    bcol = batchp.reshape(NP, 1)
    zrows = jnp.zeros((K, H), f32)
    r = lambda v: v.reshape(1, -1)

    degp = _DEG(dst1).reshape(NC, NP, 1)
    y1, dinv = _tc_in(xp, degp, W_in, r(b_in), W1)
    acc1 = _AGG(y1, src1, dst1, zrows).reshape(NC, NP, H)
    out1, st1 = _tc_post(acc1, y1, dinv, r(b1))
    y2 = _tc_mid(out1, st1, r(g1), r(be1), W2, dinv)
    acc2 = _AGG(y2, src1, dst1, zrows).reshape(NC, NP, H)
    out2, st2 = _tc_post(acc2, y2, dinv, r(b2))
    return _tc_head(out2, st2, r(g2), r(be2), brow, bcol, Wl1, r(bl1), r(gf1),
                    r(bef1), Wl2, r(bl2), r(gf2), r(bef2), Wl3, r(bl3))


# speedup vs baseline: 7.9321x; 1.2076x over previous
import functools

import jax
import jax.numpy as jnp
from jax import lax
from jax.experimental import pallas as pl
from jax.experimental.pallas import tpu as pltpu
from jax.experimental.pallas import tpu_sc as plsc

N = 10000
E = 320000
H = 128
G = 64
EPS = 1e-5

NC, NS = 2, 16
NW = NC * NS
NP = 10240
K = 128
CH = 80
RPT = NP // NS
EPW = NW * CH * K // NW
BLK = 2048
NB = NP // BLK

@functools.cache
def _mesh():
    return plsc.VectorSubcoreMesh(core_axis_name="c", subcore_axis_name="s",
                                  num_cores=NC, num_subcores=NS)


def _sc_deg_body(dst_hbm, degp_hbm, idx_v, cnt_v, red_v, wb_v, sh):
    c = lax.axis_index("c")
    s = lax.axis_index("s")
    wid = c * NS + s
    row0 = s * RPT
    ones = jnp.ones((16,), jnp.float32)
    zeros = jnp.zeros((16,), jnp.float32)

    def zstep(j, carry):
        cnt_v[pl.ds(j * 16, 16)] = zeros
        return carry

    lax.fori_loop(0, NP // 16, zstep, 0)
    pltpu.sync_copy(dst_hbm.at[pl.ds(wid * EPW, EPW)], idx_v)

    def astep(j, carry):
        idx16 = idx_v[pl.ds(j * 16, 16)]
        plsc.addupdate_scatter(cnt_v, [idx16], ones)
        return carry

    lax.fori_loop(0, EPW // 16, astep, 0)
    pltpu.sync_copy(cnt_v, sh.at[pl.ds(s * NP, NP)])
    plsc.subcore_barrier()
    for t in range(NS):
        pltpu.sync_copy(sh.at[pl.ds(t * NP + row0, RPT)], red_v.at[t])

    def rstep(j, carry):
        acc = red_v[0, pl.ds(j * 16, 16)]
        for t in range(1, NS):
            acc = acc + red_v[t, pl.ds(j * 16, 16)]
        wb_v[pl.ds(j * 16, 16)] = acc
        return carry

    lax.fori_loop(0, RPT // 16, rstep, 0)
    pltpu.sync_copy(wb_v, degp_hbm.at[pl.ds(c * NP + row0, RPT)])


def _DEG(dst1):
    return pl.kernel(
        _sc_deg_body,
        out_type=jax.ShapeDtypeStruct((NC * NP,), jnp.float32),
        mesh=_mesh(),
        compiler_params=pltpu.CompilerParams(needs_layout_passes=False),
        scratch_types=[
            pltpu.VMEM((EPW,), jnp.int32),
            pltpu.VMEM((NP,), jnp.float32),
            pltpu.VMEM((NS, RPT), jnp.float32),
            pltpu.VMEM((RPT,), jnp.float32),
            pltpu.VMEM_SHARED((NS * NP,), jnp.float32),
        ],
    )(dst1)


def _sc_agg_body(y_hbm, src_hbm, dst_hbm, zeros_hbm, accp_hbm, src0_v, src1_v,
                 dst0_v, dst1_v, rows0_v, rows1_v, acc_sh, sem0, sem1):
    c = lax.axis_index("c")
    s = lax.axis_index("s")
    wid = c * NS + s
    row0 = s * RPT
    pltpu.sync_copy(zeros_hbm, rows0_v)
    for k in range(RPT // K):
        pltpu.sync_copy(rows0_v, acc_sh.at[pl.ds(row0 + k * K, K)])
    plsc.subcore_barrier()

    srcs = (src0_v, src1_v)
    dsts = (dst0_v, dst1_v)
    rows = (rows0_v, rows1_v)
    sems = (sem0, sem1)

    def prep(j, b):
        base = (wid * CH + j) * K
        pltpu.sync_copy(src_hbm.at[pl.ds(base, K)], srcs[b])
        pltpu.sync_copy(dst_hbm.at[pl.ds(base, K)], dsts[b])
        pltpu.async_copy(y_hbm.at[srcs[b]], rows[b], sems[b])

    def drain(b):
        pltpu.make_async_copy(y_hbm.at[srcs[b]], rows[b], sems[b]).wait()
        pltpu.sync_copy(rows[b], acc_sh.at[dsts[b]], add=True)

    prep(0, 0)

    def step(g, carry):
        prep(2 * g + 1, 1)
        drain(0)

        @pl.when(g < CH // 2 - 1)
        def _():
            prep(2 * g + 2, 0)

        drain(1)
        return carry

    lax.fori_loop(0, CH // 2, step, 0)
    plsc.subcore_barrier()
    for k in range(RPT // K):
        pltpu.sync_copy(acc_sh.at[pl.ds(row0 + k * K, K)], rows0_v)
        pltpu.sync_copy(rows0_v, accp_hbm.at[pl.ds(c * NP + row0 + k * K, K)])


def _AGG(y, src1, dst1, zrows):
    return pl.kernel(
        _sc_agg_body,
        out_type=jax.ShapeDtypeStruct((NC * NP, H), jnp.float32),
        mesh=_mesh(),
        scratch_types=[
            pltpu.VMEM((K,), jnp.int32),
            pltpu.VMEM((K,), jnp.int32),
            pltpu.VMEM((K,), jnp.int32),
            pltpu.VMEM((K,), jnp.int32),
            pltpu.VMEM((K, H), jnp.float32),
            pltpu.VMEM((K, H), jnp.float32),
            pltpu.VMEM_SHARED((NP, H), jnp.float32),
            pltpu.SemaphoreType.DMA,
            pltpu.SemaphoreType.DMA,
        ],
    )(y, src1, dst1, zrows)


def _row_mask(i):
    rows = i * BLK + lax.broadcasted_iota(jnp.int32, (BLK, 1), 0)
    return (rows < N).astype(jnp.float32)


def _tc_in_body(x_ref, degp_ref, wi_ref, bi_ref, w1_ref, y_ref, dinv_ref):
    i = pl.program_id(0)
    deg = degp_ref[0] + degp_ref[1] + 1.0
    dinv = lax.rsqrt(deg)
    h0 = jnp.dot(x_ref[...], wi_ref[...],
                 preferred_element_type=jnp.float32) + bi_ref[...]
    xw = jnp.dot(h0, w1_ref[...], preferred_element_type=jnp.float32)
    y_ref[...] = xw * dinv * _row_mask(i)
    dinv_ref[...] = dinv


def _tc_in(xp, degp, w_in, b_in, w1):
    return pl.pallas_call(
        _tc_in_body,
        grid=(NB,),
        in_specs=[
            pl.BlockSpec((BLK, 3), lambda i: (i, 0)),
            pl.BlockSpec((NC, BLK, 1), lambda i: (0, i, 0)),
            pl.BlockSpec((3, H), lambda i: (0, 0)),
            pl.BlockSpec((1, H), lambda i: (0, 0)),
            pl.BlockSpec((H, H), lambda i: (0, 0)),
        ],
        out_specs=[
            pl.BlockSpec((BLK, H), lambda i: (i, 0)),
            pl.BlockSpec((BLK, 1), lambda i: (i, 0)),
        ],
        out_shape=[
            jax.ShapeDtypeStruct((NP, H), jnp.float32),
            jax.ShapeDtypeStruct((NP, 1), jnp.float32),
        ],
    )(xp, degp, w_in, b_in, w1)


def _tc_post_body(accp_ref, y_ref, dinv_ref, b_ref, out_ref, st_ref):
    i = pl.program_id(0)
    acc = accp_ref[0] + accp_ref[1] + y_ref[...]
    out = (acc * dinv_ref[...] + b_ref[...]) * _row_mask(i)
    out_ref[...] = out
    st = jnp.concatenate(
        [jnp.sum(out, 0, keepdims=True), jnp.sum(out * out, 0, keepdims=True)],
        axis=0)

    @pl.when(i == 0)
    def _():
        st_ref[...] = st

    @pl.when(i > 0)
    def _():
        st_ref[...] = st_ref[...] + st


def _tc_post(accp, y, dinv, b):
    return pl.pallas_call(
        _tc_post_body,
        grid=(NB,),
        in_specs=[
            pl.BlockSpec((NC, BLK, H), lambda i: (0, i, 0)),
            pl.BlockSpec((BLK, H), lambda i: (i, 0)),
            pl.BlockSpec((BLK, 1), lambda i: (i, 0)),
            pl.BlockSpec((1, H), lambda i: (0, 0)),
        ],
        out_specs=[
            pl.BlockSpec((BLK, H), lambda i: (i, 0)),
            pl.BlockSpec((2, H), lambda i: (0, 0)),
        ],
        out_shape=[
            jax.ShapeDtypeStruct((NP, H), jnp.float32),
            jax.ShapeDtypeStruct((2, H), jnp.float32),
        ],
    )(accp, y, dinv, b)


def _tc_mid_body(h_ref, st_ref, g_ref, be_ref, w_ref, dinv_ref, y_ref):
    i = pl.program_id(0)
    mu = st_ref[0:1] / N
    var = st_ref[1:2] / N - mu * mu
    sc = g_ref[...] * lax.rsqrt(var + EPS)
    sh = be_ref[...] - mu * sc
    h = jnp.maximum(h_ref[...] * sc + sh, 0.0)
    y = jnp.dot(h, w_ref[...], preferred_element_type=jnp.float32)
    y_ref[...] = y * dinv_ref[...] * _row_mask(i)


def _tc_mid(out1, st, g, be, w2, dinv):
    return pl.pallas_call(
        _tc_mid_body,
        grid=(NB,),
        in_specs=[
            pl.BlockSpec((BLK, H), lambda i: (i, 0)),
            pl.BlockSpec((2, H), lambda i: (0, 0)),
            pl.BlockSpec((1, H), lambda i: (0, 0)),
            pl.BlockSpec((1, H), lambda i: (0, 0)),
            pl.BlockSpec((H, H), lambda i: (0, 0)),
            pl.BlockSpec((BLK, 1), lambda i: (i, 0)),
        ],
        out_specs=pl.BlockSpec((BLK, H), lambda i: (i, 0)),
        out_shape=jax.ShapeDtypeStruct((NP, H), jnp.float32),
    )(out1, st, g, be, w2, dinv)


def _bn_rows(t, gamma, beta):
    mu = jnp.mean(t, 0, keepdims=True)
    var = jnp.mean(t * t, 0, keepdims=True) - mu * mu
    return (t - mu) * lax.rsqrt(var + EPS) * gamma + beta


def _tc_head_body(h_ref, st_ref, g_ref, be_ref, brow_ref, bcol_ref,
                  wl1_ref, bl1_ref, gf1_ref, bef1_ref,
                  wl2_ref, bl2_ref, gf2_ref, bef2_ref,
                  wl3_ref, bl3_ref, o_ref, sum_s, max_s, cnt_s):
    i = pl.program_id(0)
    mu = st_ref[0:1] / N
    var = st_ref[1:2] / N - mu * mu
    sc = g_ref[...] * lax.rsqrt(var + EPS)
    sh = be_ref[...] - mu * sc
    h = jnp.maximum(h_ref[...] * sc + sh, 0.0)

    gid = lax.broadcasted_iota(jnp.int32, (G, 1), 0)
    oht = (brow_ref[...] == gid).astype(jnp.float32)
    psum = jnp.dot(oht, h, preferred_element_type=jnp.float32)
    pcnt = jnp.sum(oht, axis=1, keepdims=True)
    bcol = bcol_ref[...]
    parts = [
        jnp.max(jnp.where(bcol == g, h, -1e30), axis=0, keepdims=True)
        for g in range(G)
    ]
    pmax = jnp.concatenate(parts, axis=0)

    @pl.when(i == 0)
    def _():
        sum_s[...] = psum
        cnt_s[...] = pcnt
        max_s[...] = pmax

    @pl.when(i > 0)
    def _():
        sum_s[...] = sum_s[...] + psum
        cnt_s[...] = cnt_s[...] + pcnt
        max_s[...] = jnp.maximum(max_s[...], pmax)

    @pl.when(i == NB - 1)
    def _():
        cnt = cnt_s[...]
        mean = sum_s[...] / jnp.maximum(cnt, 1.0)
        mx = jnp.where(cnt > 0, max_s[...], 0.0)
        hg = jnp.concatenate([mean, mx], axis=1)
        t = jnp.dot(hg, wl1_ref[...],
                    preferred_element_type=jnp.float32) + bl1_ref[...]
        t = jnp.maximum(_bn_rows(t, gf1_ref[...], bef1_ref[...]), 0.0)
        t = jnp.dot(t, wl2_ref[...],
                    preferred_element_type=jnp.float32) + bl2_ref[...]
        t = jnp.maximum(_bn_rows(t, gf2_ref[...], bef2_ref[...]), 0.0)
        o_ref[...] = jnp.dot(t, wl3_ref[...],
                             preferred_element_type=jnp.float32) + bl3_ref[...]


def _tc_head(out2, st, g, be, brow, bcol, wl1, bl1, gf1, bef1, wl2, bl2, gf2,
             bef2, wl3, bl3):
    full = lambda shape: pl.BlockSpec(shape, lambda i: tuple(0 for _ in shape))
    return pl.pallas_call(
        _tc_head_body,
        grid=(NB,),
        in_specs=[
            pl.BlockSpec((BLK, H), lambda i: (i, 0)),
            full((2, H)),
            full((1, H)),
            full((1, H)),
            pl.BlockSpec((1, BLK), lambda i: (0, i)),
            pl.BlockSpec((BLK, 1), lambda i: (i, 0)),
            full((2 * H, H)),
            full((1, H)),
            full((1, H)),
            full((1, H)),
            full((H, G)),
            full((1, G)),
            full((1, G)),
            full((1, G)),
            full((G, 2)),
            full((1, 2)),
        ],
        out_specs=full((G, 2)),
        out_shape=jax.ShapeDtypeStruct((G, 2), jnp.float32),
        scratch_shapes=[
            pltpu.VMEM((G, H), jnp.float32),
            pltpu.VMEM((G, H), jnp.float32),
            pltpu.VMEM((G, 1), jnp.float32),
        ],
    )(out2, st, g, be, brow, bcol, wl1, bl1, gf1, bef1, wl2, bl2, gf2, bef2,
      wl3, bl3)


def kernel(x, edge_index, batch, W_in, b_in, W1, b1, W2, b2, g1, be1, g2, be2,
           Wl1, bl1, gf1, bef1, Wl2, bl2, gf2, bef2, Wl3, bl3):
    f32 = jnp.float32
    pad_e = NW * CH * K - E
    src1 = jnp.concatenate([edge_index[0], jnp.full((pad_e,), N, jnp.int32)])
    dst1 = jnp.concatenate([edge_index[1], jnp.full((pad_e,), N, jnp.int32)])
    xp = jnp.zeros((NP, 3), f32).at[:N].set(x)
    batchp = jnp.concatenate([batch, jnp.full((NP - N,), G, jnp.int32)])
    brow = batchp.reshape(1, NP)
    bcol = batchp.reshape(NP, 1)
    zrows = jnp.zeros((K, H), f32)
    r = lambda v: v.reshape(1, -1)

    degp = _DEG(dst1).reshape(NC, NP, 1)
    y1, dinv = _tc_in(xp, degp, W_in, r(b_in), W1)
    acc1 = _AGG(y1, src1, dst1, zrows).reshape(NC, NP, H)
    out1, st1 = _tc_post(acc1, y1, dinv, r(b1))
    y2 = _tc_mid(out1, st1, r(g1), r(be1), W2, dinv)
    acc2 = _AGG(y2, src1, dst1, zrows).reshape(NC, NP, H)
    out2, st2 = _tc_post(acc2, y2, dinv, r(b2))
    return _tc_head(out2, st2, r(g2), r(be2), brow, bcol, Wl1, r(bl1), r(gf1),
                    r(bef1), Wl2, r(bl2), r(gf2), r(bef2), Wl3, r(bl3))
